# baseline (speedup 1.0000x reference)
import functools

import jax
import jax.numpy as jnp
from jax import lax
from jax.experimental import pallas as pl
from jax.experimental.pallas import tpu as pltpu
from jax.experimental.pallas import tpu_sc as plsc

L = 16
NC, NS = 2, 16
NW = NC * NS

BSEQ, SLEN, D = 4096, 200, 21
B = BSEQ * SLEN
NROW = B // 128
PER_W = B // NW
TILE = 8 * 128
NTASK = PER_W // TILE
GROUPS = TILE // L
NBUF = 3
NPAIR = (NTASK + 2 * NBUF - 1) // NBUF


def _make_kernel():
    mesh = plsc.VectorSubcoreMesh(core_axis_name="c", subcore_axis_name="s")

    @functools.partial(
        pl.kernel,
        mesh=mesh,
        out_type=jax.ShapeDtypeStruct((D, NROW, 128), jnp.float32),
        compiler_params=pltpu.CompilerParams(
            needs_layout_passes=False,
            skip_device_barrier=True,
            disable_bounds_checks=True,
            disable_semaphore_checks=True,
        ),
        scratch_types=[
            pltpu.VMEM((D * D,), jnp.float32),
            pltpu.VMEM((PER_W,), jnp.int32),
            pltpu.VMEM((D, 8, 128), jnp.float32),
            pltpu.VMEM((D, 8, 128), jnp.float32),
            pltpu.VMEM((D, 8, 128), jnp.float32),
            pltpu.SemaphoreType.DMA,
            pltpu.SemaphoreType.DMA,
            pltpu.SemaphoreType.DMA,
        ],
    )
    def emb(idx_hbm, table_hbm, out_hbm, table_v, idx_v, stage0, stage1,
            stage2, semo0, semo1, semo2):
        w = lax.axis_index("s") * NC + lax.axis_index("c")
        rbase = w * (PER_W // 128)
        pltpu.sync_copy(table_hbm, table_v)
        pltpu.sync_copy(idx_hbm.at[pl.ds(w * PER_W, PER_W)], idx_v)

        stages = (stage0, stage1, stage2)
        semo = (semo0, semo1, semo2)

        def pair_body(p, carry):
            for slot in range(NBUF):
                t = p * NBUF + slot

                @pl.when((t > NBUF - 1) & (t - NBUF < NTASK))
                def _drain(slot=slot):
                    pltpu.make_async_copy(
                        out_hbm.at[:, pl.ds(0, 8), :], stages[slot], semo[slot]
                    ).wait()

                @pl.when(t < NTASK)
                def _work(slot=slot, t=t):
                    s = stages[slot]

                    @plsc.parallel_loop(0, GROUPS, unroll=3)
                    def body(g):
                        idxv = idx_v[pl.ds(t * TILE + g * L, L)]
                        addr = idxv * D
                        r = lax.shift_right_logical(g, 3)
                        c0 = lax.bitwise_and(g, 7) * L
                        for k in range(D):
                            vals = plsc.load_gather(table_v, [addr + k])
                            s[k, r, pl.ds(c0, L)] = vals

                    pltpu.async_copy(
                        s,
                        out_hbm.at[:, pl.ds(rbase + t * 8, 8), :],
                        semo[slot],
                    )
            return carry

        lax.fori_loop(0, NPAIR, pair_body, 0)

    return emb


_emb = _make_kernel()


@jax.jit
def kernel(sequence_indices, table):
    NBT, NLT = BSEQ // 128, SLEN // 8
    idx_p = jnp.transpose(
        sequence_indices.reshape(NBT, 128, NLT, 8), (2, 0, 3, 1)
    ).reshape(-1)
    out_t = _emb(idx_p, table.reshape(-1))
    out = jnp.transpose(
        out_t.reshape(D, NLT, NBT, 8, 128), (2, 4, 1, 3, 0)
    ).reshape(BSEQ, SLEN, D)
    return out


# speedup vs baseline: 1.1888x; 1.1888x over previous
import functools

import jax
import jax.numpy as jnp
from jax import lax
from jax.experimental import pallas as pl
from jax.experimental.pallas import tpu as pltpu
from jax.experimental.pallas import tpu_sc as plsc

L = 16
NC, NS = 2, 16
NW = NC * NS

BSEQ, SLEN, D = 4096, 200, 21
B = BSEQ * SLEN
NROW = B // 128
PER_W = B // NW
TILE = 8 * 128
NTASK = PER_W // TILE
GROUPS = TILE // L
NBUF = 2
NPAIR = (NTASK + 2 * NBUF - 1) // NBUF


def _make_kernel():
    mesh = plsc.VectorSubcoreMesh(core_axis_name="c", subcore_axis_name="s")

    @functools.partial(
        pl.kernel,
        mesh=mesh,
        out_type=jax.ShapeDtypeStruct((D, NROW, 128), jnp.float32),
        compiler_params=pltpu.CompilerParams(needs_layout_passes=False),
        scratch_types=[
            pltpu.VMEM((D * D,), jnp.float32),
            pltpu.VMEM((PER_W,), jnp.int32),
            pltpu.VMEM((D, 8, 128), jnp.float32),
            pltpu.VMEM((D, 8, 128), jnp.float32),
            pltpu.SemaphoreType.DMA,
            pltpu.SemaphoreType.DMA,
        ],
    )
    def emb(idx_hbm, table_hbm, out_hbm, table_v, idx_v, stage0, stage1,
            semo0, semo1):
        w = lax.axis_index("s") * NC + lax.axis_index("c")
        rbase = w * (PER_W // 128)
        pltpu.sync_copy(table_hbm, table_v)
        pltpu.sync_copy(idx_hbm.at[pl.ds(w * PER_W, PER_W)], idx_v)

        stages = (stage0, stage1)
        semo = (semo0, semo1)

        def pair_body(p, carry):
            for slot in range(NBUF):
                t = p * NBUF + slot

                @pl.when((t > NBUF - 1) & (t - NBUF < NTASK))
                def _drain(slot=slot):
                    pltpu.make_async_copy(
                        out_hbm.at[:, pl.ds(0, 8), :], stages[slot], semo[slot]
                    ).wait()

                @pl.when(t < NTASK)
                def _work(slot=slot, t=t):
                    s = stages[slot]

                    @plsc.parallel_loop(0, GROUPS, unroll=2)
                    def body(g):
                        idxv = idx_v[pl.ds(t * TILE + g * L, L)]
                        addr = idxv * D
                        r = lax.shift_right_logical(g, 3)
                        c0 = lax.bitwise_and(g, 7) * L
                        for k in range(D):
                            vals = plsc.load_gather(table_v, [addr + k])
                            s[k, r, pl.ds(c0, L)] = vals

                    for k in range(D):
                        pltpu.async_copy(
                            s.at[k],
                            out_hbm.at[k, pl.ds(rbase + t * 8, 8), :],
                            semo[slot],
                        )
            return carry

        lax.fori_loop(0, NPAIR, pair_body, 0)

    return emb


_emb = _make_kernel()


@jax.jit
def kernel(sequence_indices, table):
    NBT, NLT = BSEQ // 128, SLEN // 8
    idx_p = jnp.transpose(
        sequence_indices.reshape(NBT, 128, NLT, 8), (2, 0, 3, 1)
    ).reshape(-1)
    out_t = _emb(idx_p, table.reshape(-1))
    out = jnp.transpose(
        out_t.reshape(D, NLT, NBT, 8, 128), (2, 4, 1, 3, 0)
    ).reshape(BSEQ, SLEN, D)
    return out


# speedup vs baseline: 1.2138x; 1.0210x over previous
import functools

import jax
import jax.numpy as jnp
from jax import lax
from jax.experimental import pallas as pl
from jax.experimental.pallas import tpu as pltpu
from jax.experimental.pallas import tpu_sc as plsc

L = 16
NC, NS = 2, 16
NW = NC * NS

BSEQ, SLEN, D = 4096, 200, 21
B = BSEQ * SLEN
PER_W = B // NW
TILE = 8 * 128
NTASK = PER_W // TILE
GROUPS = TILE // L
STAGE_W = D * TILE
NBUF = 2
NROUND = (NTASK + 2 * NBUF - 1) // NBUF


def _make_kernel():
    mesh = plsc.VectorSubcoreMesh(core_axis_name="c", subcore_axis_name="s")

    @functools.partial(
        pl.kernel,
        mesh=mesh,
        out_type=jax.ShapeDtypeStruct((B * D,), jnp.float32),
        compiler_params=pltpu.CompilerParams(needs_layout_passes=False),
        scratch_types=[
            pltpu.VMEM((D * D,), jnp.float32),
            pltpu.VMEM((PER_W,), jnp.int32),
            pltpu.VMEM((STAGE_W,), jnp.float32),
            pltpu.VMEM((STAGE_W,), jnp.float32),
            pltpu.SemaphoreType.DMA,
            pltpu.SemaphoreType.DMA,
        ],
    )
    def emb(idx_hbm, table_hbm, out_hbm, table_v, idx_v, stage0, stage1,
            semo0, semo1):
        w = lax.axis_index("s") * NC + lax.axis_index("c")
        nbase = w * PER_W
        pltpu.sync_copy(table_hbm, table_v)
        pltpu.sync_copy(idx_hbm.at[pl.ds(nbase, PER_W)], idx_v)

        stages = (stage0, stage1)
        semo = (semo0, semo1)

        def round_body(p, carry):
            for slot in range(NBUF):
                t = p * NBUF + slot

                @pl.when((t > NBUF - 1) & (t - NBUF < NTASK))
                def _drain(slot=slot):
                    pltpu.make_async_copy(
                        out_hbm.at[pl.ds(0, STAGE_W)], stages[slot], semo[slot]
                    ).wait()

                @pl.when(t < NTASK)
                def _work(slot=slot, t=t):
                    s = stages[slot]

                    @plsc.parallel_loop(0, GROUPS, unroll=2)
                    def body(g):
                        idxv = idx_v[pl.ds(t * TILE + g * L, L)]
                        addr = idxv * D
                        for k in range(D):
                            vals = plsc.load_gather(table_v, [addr + k])
                            s[pl.ds(k * TILE + g * L, L)] = vals

                    for k in range(D):
                        pltpu.async_copy(
                            s.at[pl.ds(k * TILE, TILE)],
                            out_hbm.at[pl.ds(k * B + nbase + t * TILE, TILE)],
                            semo[slot],
                        )
            return carry

        lax.fori_loop(0, NROUND, round_body, 0)

    return emb


_emb = _make_kernel()


@jax.jit
def kernel(sequence_indices, table):
    NBT, NLT = BSEQ // 128, SLEN // 8
    idx_p = jnp.transpose(
        sequence_indices.reshape(NBT, 128, NLT, 8), (2, 0, 3, 1)
    ).reshape(-1)
    out_flat = _emb(idx_p, table.reshape(-1))
    out = jnp.transpose(
        out_flat.reshape(D, NLT, NBT, 8, 128), (2, 4, 1, 3, 0)
    ).reshape(BSEQ, SLEN, D)
    return out
